# baseline (speedup 1.0000x reference)
import jax
import jax.numpy as jnp
from jax.experimental import pallas as pl
from jax.experimental.pallas import tpu as pltpu

C = 256
H = 256
FEAT = 256
T = 4
ITER = 2
NUM_SEM = 57

BI = 32
GI = C // BI
NPAR = 32
BCOL = (H * C) // NPAR

P_PREP = NPAR
P_T0 = NPAR + 1
P_UPD = P_T0 + GI
P_T1 = P_UPD + 1
P_HEAD = P_T1 + GI
NPHASE = P_HEAD + 1

f32 = jnp.float32


def _mega_kernel(parent_ref, wp_ref, bp_ref, wex_ref, bex_ref,
                 wela_ref, welb_ref, bel_ref,
                 w1_ref, w2_ref, bne_ref, weet_ref, bee_ref, w3_ref, w4_ref,
                 wc0_ref, wc1_ref, wc2_ref, bc_ref,
                 wsem_ref, bsem_ref, wch2_ref, bch2_ref,
                 out_out, sem_out, cel_out, eel_out, cnt_out,
                 cf_s, a_s, b_s, u_s, v_s, cf1_s, agg_s):
    p = pl.program_id(0)

    @pl.when(p < NPAR)
    def _():
        pf = jnp.dot(parent_ref[:, :], wp_ref[:, :], preferred_element_type=f32)
        blk = jnp.maximum(pf + bp_ref[:, :], 0.0).reshape(BCOL // H, H)
        cf_s[pl.ds(p * (BCOL // H), BCOL // H), :] = blk

    @pl.when(p == P_PREP)
    def _():
        cf = cf_s[:, :]
        cel_out[:, :] = jnp.dot(cf, wex_ref[:, :], preferred_element_type=f32) + bex_ref[:, :]
        a_s[:, :] = jnp.dot(cf, wela_ref[:, :], preferred_element_type=f32)
        b_s[:, :] = jnp.dot(cf, welb_ref[:, :], preferred_element_type=f32) + bel_ref[:, :]
        u_s[:, :] = jnp.dot(cf, w1_ref[:H, :], preferred_element_type=f32) + bne_ref[0:1, :]
        v_s[:, :] = jnp.dot(cf, w2_ref[:H, :], preferred_element_type=f32)

    @pl.when(p == P_UPD)
    def _():
        has_edges = cnt_out[0, 0] > 0.0
        cf1 = jnp.where(has_edges, agg_s[:, :], cf_s[:, :])
        cf1_s[:, :] = cf1
        u_s[:, :] = jnp.dot(cf1, w1_ref[H:, :], preferred_element_type=f32) + bne_ref[1:2, :]
        v_s[:, :] = jnp.dot(cf1, w2_ref[H:, :], preferred_element_type=f32)

    is_t0 = (p >= P_T0) & (p < P_T0 + GI)
    is_t1 = (p >= P_T1) & (p < P_T1 + GI)

    @pl.when(is_t0 | is_t1)
    def _():
        it = (p >= P_T1).astype(jnp.int32)
        gi = p - P_T0 - it * (GI + 1)

        a = a_s[pl.ds(gi * BI, BI), :]
        b = b_s[:, :]
        u = u_s[pl.ds(gi * BI, BI), :]
        v = v_s[:, :]
        ci = cel_out[pl.ds(gi * BI, BI), :]
        cj = cel_out[:, :]
        w3 = w3_ref[pl.ds(it * H, H), :]
        w4 = w4_ref[it, :, :]

        el = jnp.maximum(a[:, None, :] + b[None, :, :], 0.0)
        el2 = el.reshape(BI * C, H)

        eel = jnp.dot(el2, weet_ref[:, :], preferred_element_type=f32) + bee_ref[:, :]
        eel3 = eel.reshape(BI, C, T)

        cif = (ci > 0.0).astype(f32)
        cjf = (cj > 0.0).astype(f32)
        maskf = (eel3 > 0.0).astype(f32) * cif[:, :, None] * cjf[None, :, :]

        el3m = jnp.dot(el2, w3, preferred_element_type=f32)
        base = el3m.reshape(BI, C, H) + u[:, None, :] + v[None, :, :]

        msg = jnp.zeros((BI, C, H), dtype=f32)
        for t in range(T):
            w4t = w4[t:t + 1, :]
            contrib = jnp.maximum(base + eel3[:, :, t:t + 1] * w4t[None, :, :], 0.0)
            msg = msg + contrib * maskf[:, :, t:t + 1]

        agg_s[pl.ds(gi * BI, BI), :] = jnp.sum(msg, axis=1)

        @pl.when(p < P_UPD)
        def _():
            eel_out[:, :] = eel

            @pl.when(gi == 0)
            def _():
                cnt_out[:, :] = jnp.sum(maskf).reshape(1, 1)

            @pl.when(gi != 0)
            def _():
                cnt_out[:, :] = cnt_out[:, :] + jnp.sum(maskf).reshape(1, 1)

    @pl.when(p == P_HEAD)
    def _():
        has_edges = cnt_out[0, 0] > 0.0
        cf2 = jnp.where(has_edges, agg_s[:, :], cf1_s[:, :])
        hid = (jnp.dot(cf_s[:, :], wc0_ref[:, :], preferred_element_type=f32)
               + jnp.dot(cf1_s[:, :], wc1_ref[:, :], preferred_element_type=f32)
               + jnp.dot(cf2, wc2_ref[:, :], preferred_element_type=f32)
               + bc_ref[:, :])
        hid = jnp.maximum(hid, 0.0)
        sem_out[:, :] = jnp.dot(hid, wsem_ref[:, :], preferred_element_type=f32) + bsem_ref[:, :]
        out_out[:, :] = jnp.maximum(
            jnp.dot(hid, wch2_ref[:, :], preferred_element_type=f32) + bch2_ref[:, :], 0.0)


def _full(shape):
    return pl.BlockSpec(shape, lambda p: tuple(0 for _ in shape))


@jax.jit
def kernel(parent_feature, W_parent, b_parent, W_exists, b_exists, W_el, b_el,
           W_ee, b_ee, W_ne, b_ne, W_child, b_child, W_sem, b_sem, W_child2, b_child2):
    W1 = jnp.concatenate([W_ne[0, :H], W_ne[1, :H]], axis=0)
    W2 = jnp.concatenate([W_ne[0, H:2 * H], W_ne[1, H:2 * H]], axis=0)
    W3 = jnp.concatenate([W_ne[0, 2 * H:3 * H], W_ne[1, 2 * H:3 * H]], axis=0)
    W4 = jnp.stack([W_ne[0, 3 * H:], W_ne[1, 3 * H:]], axis=0)

    out_feats, sem, cel, eel, _cnt = pl.pallas_call(
        _mega_kernel,
        grid=(NPHASE,),
        in_specs=[
            _full((1, FEAT)),
            pl.BlockSpec((FEAT, BCOL), lambda p: (0, jnp.minimum(p, NPAR - 1))),
            pl.BlockSpec((1, BCOL), lambda p: (0, jnp.minimum(p, NPAR - 1))),
            _full((H, 1)), _full((1, 1)),
            _full((H, H)), _full((H, H)), _full((1, H)),
            _full((2 * H, H)), _full((2 * H, H)), _full((2, H)),
            _full((H, T)), _full((1, T)), _full((2 * H, H)), _full((2, T, H)),
            _full((H, H)), _full((H, H)), _full((H, H)), _full((1, H)),
            _full((H, NUM_SEM)), _full((1, NUM_SEM)), _full((H, FEAT)), _full((1, FEAT)),
        ],
        out_specs=[
            _full((C, FEAT)), _full((C, NUM_SEM)), _full((C, 1)),
            pl.BlockSpec((BI * C, T), lambda p: (jnp.clip(p - P_T0, 0, GI - 1), 0)),
            _full((1, 1)),
        ],
        out_shape=[
            jax.ShapeDtypeStruct((C, FEAT), f32),
            jax.ShapeDtypeStruct((C, NUM_SEM), f32),
            jax.ShapeDtypeStruct((C, 1), f32),
            jax.ShapeDtypeStruct((C * C, T), f32),
            jax.ShapeDtypeStruct((1, 1), f32),
        ],
        scratch_shapes=[pltpu.VMEM((C, H), f32)] * 7,
    )(parent_feature, W_parent, b_parent.reshape(1, H * C),
      W_exists, b_exists.reshape(1, 1), W_el[:H], W_el[H:], b_el.reshape(1, H),
      W1, W2, b_ne, W_ee.T, b_ee.reshape(1, T), W3, W4,
      W_child[:H], W_child[H:2 * H], W_child[2 * H:], b_child.reshape(1, H),
      W_sem, b_sem.reshape(1, NUM_SEM), W_child2, b_child2.reshape(1, FEAT))

    return (out_feats.reshape(1, C, FEAT),
            sem.reshape(1, C, NUM_SEM),
            cel.reshape(1, C, 1),
            eel.reshape(1, C, C, T))


# speedup vs baseline: 1.4477x; 1.4477x over previous
import jax
import jax.numpy as jnp
from jax.experimental import pallas as pl
from jax.experimental.pallas import tpu as pltpu

C = 256
H = 256
FEAT = 256
T = 4
ITER = 2
NUM_SEM = 57

BI = 16
GI = C // BI
NPAR = 16
BCOL = (H * C) // NPAR

f32 = jnp.float32


def _edge_tile(gi, a_full, b_full, u_full, v_full, cel_full,
               weet_ref, bee_ref, w3_ref, w4_ref):
    a = a_full[pl.ds(gi * BI, BI), :]
    b = b_full[:, :]
    u = u_full[pl.ds(gi * BI, BI), :]
    v = v_full[:, :]
    ci = cel_full[pl.ds(gi * BI, BI), :]
    cj = cel_full[:, :]

    el = jnp.maximum(a[:, None, :] + b[None, :, :], 0.0)
    el2 = el.reshape(BI * C, H)

    eel = jnp.dot(el2, weet_ref[:, :], preferred_element_type=f32) + bee_ref[:, :]
    eel3 = eel.reshape(BI, C, T)

    cif = (ci > 0.0).astype(f32)
    cjf = (cj > 0.0).astype(f32)
    maskf = (eel3 > 0.0).astype(f32) * cif[:, :, None] * cjf[None, :, :]

    el3m = jnp.dot(el2, w3_ref[:, :], preferred_element_type=f32)
    base = el3m.reshape(BI, C, H) + u[:, None, :] + v[None, :, :]

    bf16 = jnp.bfloat16
    base16 = base.astype(bf16)
    eel16 = eel3.astype(bf16)
    mask16 = maskf.astype(bf16)
    w416 = w4_ref[:, :].astype(bf16)
    msg = jnp.zeros((BI, C, H), dtype=bf16)
    for t in range(T):
        w4t = w416[t:t + 1, :]
        contrib = jnp.maximum(base16 + eel16[:, :, t:t + 1] * w4t[None, :, :], 0)
        msg = msg + contrib * mask16[:, :, t:t + 1]

    tile_agg = jnp.sum(msg.astype(f32), axis=1)
    tile_cnt = jnp.sum(maskf).reshape(1, 1)
    return eel, tile_agg, tile_cnt


def _phase_a_kernel(parent_ref, wp_ref, bp_ref, wex_ref, bex_ref,
                    wela_ref, welb_ref, bel_ref, w1_ref, w2_ref, bne_ref,
                    weet_ref, bee_ref, w3_ref, w4_ref,
                    cf_out, cel_out, a_out, b_out, eel_out, agg_out, cnt_out,
                    cf_s, u_s, v_s):
    p = pl.program_id(0)

    @pl.when(p < NPAR)
    def _():
        pf = jnp.dot(parent_ref[:, :], wp_ref[:, :], preferred_element_type=f32)
        blk = jnp.maximum(pf + bp_ref[:, :], 0.0).reshape(BCOL // H, H)
        cf_s[pl.ds(p * (BCOL // H), BCOL // H), :] = blk

    @pl.when(p == NPAR)
    def _():
        cf = cf_s[:, :]
        cf_out[:, :] = cf
        cel_out[:, :] = jnp.dot(cf, wex_ref[:, :], preferred_element_type=f32) + bex_ref[:, :]
        a_out[:, :] = jnp.dot(cf, wela_ref[:, :], preferred_element_type=f32)
        b_out[:, :] = jnp.dot(cf, welb_ref[:, :], preferred_element_type=f32) + bel_ref[:, :]
        u_s[:, :] = jnp.dot(cf, w1_ref[:, :], preferred_element_type=f32) + bne_ref[:, :]
        v_s[:, :] = jnp.dot(cf, w2_ref[:, :], preferred_element_type=f32)

    @pl.when(p > NPAR)
    def _():
        gi = p - (NPAR + 1)
        eel, tile_agg, tile_cnt = _edge_tile(
            gi, a_out, b_out, u_s, v_s, cel_out,
            weet_ref, bee_ref, w3_ref, w4_ref)
        eel_out[:, :] = eel
        agg_out[pl.ds(gi * BI, BI), :] = tile_agg

        @pl.when(gi == 0)
        def _():
            cnt_out[:, :] = tile_cnt

        @pl.when(gi != 0)
        def _():
            cnt_out[:, :] = cnt_out[:, :] + tile_cnt


def _phase_b_kernel(cf0_ref, cel_ref, a_ref, b_ref, agg0_ref, cnt_ref,
                    w1_ref, w2_ref, bne_ref, weet_ref, bee_ref, w3_ref, w4_ref,
                    wc0_ref, wc1_ref, wc2_ref, bc_ref,
                    wsem_ref, bsem_ref, wch2_ref, bch2_ref,
                    out_out, sem_out,
                    cf1_s, u_s, v_s, agg1_s):
    p = pl.program_id(0)

    @pl.when(p == 0)
    def _():
        has_edges = cnt_ref[0, 0] > 0.0
        cf1 = jnp.where(has_edges, agg0_ref[:, :], cf0_ref[:, :])
        cf1_s[:, :] = cf1
        u_s[:, :] = jnp.dot(cf1, w1_ref[:, :], preferred_element_type=f32) + bne_ref[:, :]
        v_s[:, :] = jnp.dot(cf1, w2_ref[:, :], preferred_element_type=f32)

    @pl.when((p > 0) & (p <= GI))
    def _():
        gi = p - 1
        _, tile_agg, _ = _edge_tile(
            gi, a_ref, b_ref, u_s, v_s, cel_ref,
            weet_ref, bee_ref, w3_ref, w4_ref)
        agg1_s[pl.ds(gi * BI, BI), :] = tile_agg

    @pl.when(p == GI + 1)
    def _():
        has_edges = cnt_ref[0, 0] > 0.0
        cf2 = jnp.where(has_edges, agg1_s[:, :], cf1_s[:, :])
        hid = (jnp.dot(cf0_ref[:, :], wc0_ref[:, :], preferred_element_type=f32)
               + jnp.dot(cf1_s[:, :], wc1_ref[:, :], preferred_element_type=f32)
               + jnp.dot(cf2, wc2_ref[:, :], preferred_element_type=f32)
               + bc_ref[:, :])
        hid = jnp.maximum(hid, 0.0)
        sem_out[:, :] = jnp.dot(hid, wsem_ref[:, :], preferred_element_type=f32) + bsem_ref[:, :]
        out_out[:, :] = jnp.maximum(
            jnp.dot(hid, wch2_ref[:, :], preferred_element_type=f32) + bch2_ref[:, :], 0.0)


def _full(shape):
    return pl.BlockSpec(shape, lambda p: tuple(0 for _ in shape))


@jax.jit
def kernel(parent_feature, W_parent, b_parent, W_exists, b_exists, W_el, b_el,
           W_ee, b_ee, W_ne, b_ne, W_child, b_child, W_sem, b_sem, W_child2, b_child2):
    W_el_a = W_el[:H]
    W_el_b = W_el[H:]
    W1 = [W_ne[i, :H] for i in range(ITER)]
    W2 = [W_ne[i, H:2 * H] for i in range(ITER)]
    W3 = [W_ne[i, 2 * H:3 * H] for i in range(ITER)]
    W4 = [W_ne[i, 3 * H:] for i in range(ITER)]
    bne = [b_ne[i].reshape(1, H) for i in range(ITER)]
    WeeT = W_ee.T

    cf0, cel, A, B, eel, agg0, cnt = pl.pallas_call(
        _phase_a_kernel,
        grid=(NPAR + 1 + GI,),
        in_specs=[
            _full((1, FEAT)),
            pl.BlockSpec((FEAT, BCOL), lambda p: (0, jnp.minimum(p, NPAR - 1))),
            pl.BlockSpec((1, BCOL), lambda p: (0, jnp.minimum(p, NPAR - 1))),
            _full((H, 1)), _full((1, 1)),
            _full((H, H)), _full((H, H)), _full((1, H)),
            _full((H, H)), _full((H, H)), _full((1, H)),
            _full((H, T)), _full((1, T)), _full((H, H)), _full((T, H)),
        ],
        out_specs=[
            _full((C, H)), _full((C, 1)), _full((C, H)), _full((C, H)),
            pl.BlockSpec((BI * C, T), lambda p: (jnp.maximum(p - (NPAR + 1), 0), 0)),
            _full((C, H)), _full((1, 1)),
        ],
        out_shape=[
            jax.ShapeDtypeStruct((C, H), f32),
            jax.ShapeDtypeStruct((C, 1), f32),
            jax.ShapeDtypeStruct((C, H), f32),
            jax.ShapeDtypeStruct((C, H), f32),
            jax.ShapeDtypeStruct((C * C, T), f32),
            jax.ShapeDtypeStruct((C, H), f32),
            jax.ShapeDtypeStruct((1, 1), f32),
        ],
        scratch_shapes=[pltpu.VMEM((C, H), f32)] * 3,
    )(parent_feature, W_parent, b_parent.reshape(1, H * C),
      W_exists, b_exists.reshape(1, 1), W_el_a, W_el_b, b_el.reshape(1, H),
      W1[0], W2[0], bne[0], WeeT, b_ee.reshape(1, T), W3[0], W4[0])

    out_feats, sem = pl.pallas_call(
        _phase_b_kernel,
        grid=(1 + GI + 1,),
        in_specs=[
            _full((C, H)), _full((C, 1)), _full((C, H)), _full((C, H)),
            _full((C, H)), _full((1, 1)),
            _full((H, H)), _full((H, H)), _full((1, H)),
            _full((H, T)), _full((1, T)), _full((H, H)), _full((T, H)),
            _full((H, H)), _full((H, H)), _full((H, H)), _full((1, H)),
            _full((H, NUM_SEM)), _full((1, NUM_SEM)), _full((H, FEAT)), _full((1, FEAT)),
        ],
        out_specs=[_full((C, FEAT)), _full((C, NUM_SEM))],
        out_shape=[
            jax.ShapeDtypeStruct((C, FEAT), f32),
            jax.ShapeDtypeStruct((C, NUM_SEM), f32),
        ],
        scratch_shapes=[pltpu.VMEM((C, H), f32)] * 4,
    )(cf0, cel, A, B, agg0, cnt,
      W1[1], W2[1], bne[1], WeeT, b_ee.reshape(1, T), W3[1], W4[1],
      W_child[:H], W_child[H:2 * H], W_child[2 * H:], b_child.reshape(1, H),
      W_sem, b_sem.reshape(1, NUM_SEM), W_child2, b_child2.reshape(1, FEAT))

    return (out_feats.reshape(1, C, FEAT),
            sem.reshape(1, C, NUM_SEM),
            cel.reshape(1, C, 1),
            eel.reshape(1, C, C, T))
